# baseline (speedup 1.0000x reference)
import numpy as np

import jax
import jax.numpy as jnp
from jax.experimental import pallas as pl
from jax.experimental.pallas import tpu as pltpu

_TB = 512


def _sig(v):
    return 0.5 * jnp.tanh(0.5 * v) + 0.5


def _conv1_maps():
    idx = np.zeros((768, 256), np.int32)
    msk = np.zeros((768, 256), bool)
    for dh in range(4):
        for p in range(2):
            for co in range(6):
                for wh in range(14):
                    row = dh * 192 + p * 96 + co * 16 + wh
                    for dy in range(5):
                        for dx in range(5):
                            col = (dh + dy) * 32 + (2 * wh + p + dx)
                            idx[row, col] = co * 25 + dy * 5 + dx
                            msk[row, col] = True
    co_of_row = np.zeros((768,), np.int32)
    for dh in range(4):
        for p in range(2):
            for co in range(6):
                for wh in range(16):
                    co_of_row[dh * 192 + p * 96 + co * 16 + wh] = co
    return idx, msk, co_of_row


def _conv2_maps():
    idx = np.zeros((320, 576), np.int32)
    msk = np.zeros((320, 576), bool)
    for dh in range(2):
        for p in range(2):
            for co in range(16):
                for wp in range(5):
                    row = dh * 160 + p * 80 + co * 5 + wp
                    for dy in range(5):
                        for ci in range(6):
                            for dx in range(5):
                                col = (dh + dy) * 96 + ci * 16 + (2 * wp + p + dx)
                                idx[row, col] = ((co * 6 + ci) * 5 + dy) * 5 + dx
                                msk[row, col] = True
    co_of_row = np.zeros((320,), np.int32)
    for dh in range(2):
        for p in range(2):
            for co in range(16):
                for wp in range(5):
                    co_of_row[dh * 160 + p * 80 + co * 5 + wp] = co
    return idx, msk, co_of_row


_A1_IDX, _A1_MSK, _B1_ROW = _conv1_maps()
_A2_IDX, _A2_MSK, _B2_ROW = _conv2_maps()


def _shift_rows(base, nrows, width):
    tiled = jnp.tile(base, (1,) * (base.ndim - 1) + (nrows,))
    return tiled[..., : nrows * width].reshape(*base.shape[:-1], nrows, width)


def _build_a1(w1):
    f32 = jnp.float32
    w = w1.reshape(6, 5, 5).astype(f32)
    parts = []
    for p in range(2):
        base = jnp.pad(w, ((0, 0), (0, 0), (p, 29 - p)))
        t = _shift_rows(base, 16, 32)
        parts.append(t)
    t1 = jnp.stack(parts, axis=0)
    whm = jnp.asarray((np.arange(16) < 14).astype(np.float32)).reshape(1, 1, 1, 16, 1)
    t1 = t1 * whm
    t1 = jnp.transpose(t1, (0, 1, 3, 4, 2))
    t1 = jnp.pad(t1, ((0, 0),) * 4 + ((0, 4),))
    t1 = _shift_rows(t1, 4, 8)
    t1 = jnp.transpose(t1, (4, 0, 1, 2, 5, 3))
    return t1.reshape(768, 256)


def _build_a2(w2):
    f32 = jnp.float32
    w = w2.astype(f32)
    parts = []
    for p in range(2):
        base = jnp.pad(w, ((0, 0),) * 3 + ((p, 13 - p),))
        t = _shift_rows(base, 5, 16)
        parts.append(t)
    t2 = jnp.stack(parts, axis=0)
    t2 = jnp.transpose(t2, (0, 1, 2, 4, 5, 3))
    t2 = jnp.pad(t2, ((0, 0),) * 5 + ((0, 2),))
    t2 = _shift_rows(t2, 2, 6)
    t2 = jnp.transpose(t2, (5, 0, 1, 3, 6, 2, 4))
    return 0.25 * t2.reshape(320, 576)


def _lenet_body(x_ref, a1_ref, b1_ref, a2_ref, b2_ref,
                w3_ref, b3_ref, w4_ref, b4_ref, w5_ref, b5_ref,
                out_ref, p1_ref, feat_ref):
    f32 = jnp.float32
    bf16 = jnp.bfloat16

    for c in range(7):
        xs = x_ref[:, 128 * c:128 * c + 256]
        acc = jax.lax.dot_general(
            a1_ref[...], xs, (((1,), (1,)), ((), ())),
            preferred_element_type=f32)
        s = jnp.tanh(acc + b1_ref[...])
        s4 = s.reshape(2, 2, 2, 96, _TB)
        pooled = s4[:, 0, 0] + s4[:, 0, 1] + s4[:, 1, 0] + s4[:, 1, 1]
        p1_ref[192 * c:192 * c + 192, :] = (
            pooled.reshape(192, _TB).astype(bf16))

    for c in range(5):
        ps = p1_ref[192 * c:192 * c + 576, :]
        acc = jnp.dot(a2_ref[...], ps, preferred_element_type=f32)
        s = jnp.tanh(acc + b2_ref[...])
        s4 = s.reshape(2, 2, 80, _TB)
        pooled = s4[0, 0] + s4[0, 1] + s4[1, 0] + s4[1, 1]
        feat_ref[80 * c:80 * c + 80, :] = pooled.astype(bf16)

    ft = feat_ref[...]
    h1 = jnp.tanh(jnp.dot(w3_ref[...], ft, preferred_element_type=f32) + b3_ref[...])
    h2 = jnp.tanh(jnp.dot(w4_ref[...], h1.astype(bf16),
                          preferred_element_type=f32) + b4_ref[...])
    out_ref[...] = jnp.dot(w5_ref[...], h2.astype(bf16),
                           preferred_element_type=f32) + b5_ref[...]


def kernel(x, w1, b1, w2, b2, w3, b3, w4, b4, w5, b5):
    f32, bf16 = jnp.float32, jnp.bfloat16
    tb = _TB
    B = int(np.prod(x.shape)) // 784
    G = pl.cdiv(B, tb)
    Bp = G * tb

    x2 = x.reshape(B, 28, 28).astype(bf16)
    x4 = jnp.pad(x2, ((0, Bp - B), (2, 2), (2, 2))).reshape(Bp, 1024)

    a1 = (0.5 * _build_a1(w1)).astype(bf16)
    b1r = jnp.broadcast_to((0.5 * b1).astype(f32).reshape(1, 1, 6, 1),
                           (4, 2, 6, 16)).reshape(768, 1)
    a2 = (0.25 * _build_a2(w2)).astype(bf16)
    b2v = 0.5 * (b2 + 0.5 * w2.sum(axis=(1, 2, 3)))
    b2r = jnp.broadcast_to(b2v.astype(f32).reshape(1, 1, 16, 1),
                           (2, 2, 16, 5)).reshape(320, 1)

    w3n = (0.0625 * jnp.transpose(w3.reshape(120, 16, 5, 5).astype(f32),
                                  (0, 2, 1, 3)).reshape(120, 400)).astype(bf16)
    b3m = (0.5 * b3 + 0.25 * w3.sum(axis=1)).reshape(120, 1).astype(f32)
    w4b = (0.25 * w4).astype(bf16)
    b4m = (0.5 * b4 + 0.25 * w4.sum(axis=1)).reshape(84, 1).astype(f32)
    w5b = (0.5 * w5).astype(bf16)
    b5m = (b5 + 0.5 * w5.sum(axis=1)).reshape(10, 1).astype(f32)

    def vmem_const(a):
        return pl.BlockSpec(a.shape, lambda g, nd=a.ndim: (0,) * nd)

    out = pl.pallas_call(
        _lenet_body,
        out_shape=jax.ShapeDtypeStruct((10, Bp), f32),
        grid=(G,),
        in_specs=[
            pl.BlockSpec((tb, 1024), lambda g: (g, 0)),
            vmem_const(a1), vmem_const(b1r),
            vmem_const(a2), vmem_const(b2r),
            vmem_const(w3n), vmem_const(b3m),
            vmem_const(w4b), vmem_const(b4m),
            vmem_const(w5b), vmem_const(b5m),
        ],
        out_specs=pl.BlockSpec((10, tb), lambda g: (0, g)),
        scratch_shapes=[
            pltpu.VMEM((1344, tb), bf16),
            pltpu.VMEM((400, tb), bf16),
        ],
        compiler_params=pltpu.CompilerParams(
            dimension_semantics=("parallel",),
            vmem_limit_bytes=48 * 1024 * 1024,
        ),
    )(x4, a1, b1r, a2, b2r, w3n, b3m, w4b, b4m, w5b, b5m)

    return out[:, :B].T


# speedup vs baseline: 1.0756x; 1.0756x over previous
import numpy as np

import jax
import jax.numpy as jnp
from jax.experimental import pallas as pl
from jax.experimental.pallas import tpu as pltpu

_TB = 2048


def _sig(v):
    return 0.5 * jnp.tanh(0.5 * v) + 0.5


def _conv1_maps():
    idx = np.zeros((768, 256), np.int32)
    msk = np.zeros((768, 256), bool)
    for dh in range(4):
        for p in range(2):
            for co in range(6):
                for wh in range(14):
                    row = dh * 192 + p * 96 + co * 16 + wh
                    for dy in range(5):
                        for dx in range(5):
                            col = (dh + dy) * 32 + (2 * wh + p + dx)
                            idx[row, col] = co * 25 + dy * 5 + dx
                            msk[row, col] = True
    co_of_row = np.zeros((768,), np.int32)
    for dh in range(4):
        for p in range(2):
            for co in range(6):
                for wh in range(16):
                    co_of_row[dh * 192 + p * 96 + co * 16 + wh] = co
    return idx, msk, co_of_row


def _conv2_maps():
    idx = np.zeros((320, 576), np.int32)
    msk = np.zeros((320, 576), bool)
    for dh in range(2):
        for p in range(2):
            for co in range(16):
                for wp in range(5):
                    row = dh * 160 + p * 80 + co * 5 + wp
                    for dy in range(5):
                        for ci in range(6):
                            for dx in range(5):
                                col = (dh + dy) * 96 + ci * 16 + (2 * wp + p + dx)
                                idx[row, col] = ((co * 6 + ci) * 5 + dy) * 5 + dx
                                msk[row, col] = True
    co_of_row = np.zeros((320,), np.int32)
    for dh in range(2):
        for p in range(2):
            for co in range(16):
                for wp in range(5):
                    co_of_row[dh * 160 + p * 80 + co * 5 + wp] = co
    return idx, msk, co_of_row


_A1_IDX, _A1_MSK, _B1_ROW = _conv1_maps()
_A2_IDX, _A2_MSK, _B2_ROW = _conv2_maps()


def _shift_rows(base, nrows, width):
    tiled = jnp.tile(base, (1,) * (base.ndim - 1) + (nrows,))
    return tiled[..., : nrows * width].reshape(*base.shape[:-1], nrows, width)


def _build_a1(w1):
    f32 = jnp.float32
    w = w1.reshape(6, 5, 5).astype(f32)
    parts = []
    for p in range(2):
        base = jnp.pad(w, ((0, 0), (0, 0), (p, 29 - p)))
        t = _shift_rows(base, 16, 32)
        parts.append(t)
    t1 = jnp.stack(parts, axis=0)
    whm = jnp.asarray((np.arange(16) < 14).astype(np.float32)).reshape(1, 1, 1, 16, 1)
    t1 = t1 * whm
    t1 = jnp.transpose(t1, (0, 1, 3, 4, 2))
    t1 = jnp.pad(t1, ((0, 0),) * 4 + ((0, 4),))
    t1 = _shift_rows(t1, 4, 8)
    t1 = jnp.transpose(t1, (4, 0, 1, 2, 5, 3))
    return t1.reshape(768, 256)


def _build_a2(w2):
    f32 = jnp.float32
    w = w2.astype(f32)
    parts = []
    for p in range(2):
        base = jnp.pad(w, ((0, 0),) * 3 + ((p, 13 - p),))
        t = _shift_rows(base, 5, 16)
        parts.append(t)
    t2 = jnp.stack(parts, axis=0)
    t2 = jnp.transpose(t2, (0, 1, 2, 4, 5, 3))
    t2 = jnp.pad(t2, ((0, 0),) * 5 + ((0, 2),))
    t2 = _shift_rows(t2, 2, 6)
    t2 = jnp.transpose(t2, (5, 0, 1, 3, 6, 2, 4))
    return 0.25 * t2.reshape(320, 576)


def _lenet_body(x_ref, a1_ref, b1_ref, a2_ref, b2_ref,
                w3_ref, b3_ref, w4_ref, b4_ref, w5_ref, b5_ref,
                out_ref, p1_ref, feat_ref):
    f32 = jnp.float32
    bf16 = jnp.bfloat16

    for c in range(7):
        xs = x_ref[:, 128 * c:128 * c + 256]
        acc = jax.lax.dot_general(
            a1_ref[...], xs, (((1,), (1,)), ((), ())),
            preferred_element_type=f32)
        s = jnp.tanh(acc + b1_ref[...])
        s4 = s.reshape(2, 2, 2, 96, _TB)
        pooled = s4[:, 0, 0] + s4[:, 0, 1] + s4[:, 1, 0] + s4[:, 1, 1]
        p1_ref[192 * c:192 * c + 192, :] = (
            pooled.reshape(192, _TB).astype(bf16))

    for c in range(5):
        ps = p1_ref[192 * c:192 * c + 576, :]
        acc = jnp.dot(a2_ref[...], ps, preferred_element_type=f32)
        s = jnp.tanh(acc + b2_ref[...])
        s4 = s.reshape(2, 2, 80, _TB)
        pooled = s4[0, 0] + s4[0, 1] + s4[1, 0] + s4[1, 1]
        feat_ref[80 * c:80 * c + 80, :] = pooled.astype(bf16)

    ft = feat_ref[...]
    h1 = jnp.tanh(jnp.dot(w3_ref[...], ft, preferred_element_type=f32) + b3_ref[...])
    h2 = jnp.tanh(jnp.dot(w4_ref[...], h1.astype(bf16),
                          preferred_element_type=f32) + b4_ref[...])
    out_ref[...] = jnp.dot(w5_ref[...], h2.astype(bf16),
                           preferred_element_type=f32) + b5_ref[...]


def kernel(x, w1, b1, w2, b2, w3, b3, w4, b4, w5, b5):
    f32, bf16 = jnp.float32, jnp.bfloat16
    tb = _TB
    B = int(np.prod(x.shape)) // 784
    G = pl.cdiv(B, tb)
    Bp = G * tb

    x2 = x.reshape(B, 28, 28).astype(bf16)
    x4 = jnp.pad(x2, ((0, Bp - B), (2, 2), (2, 2))).reshape(Bp, 1024)

    a1 = (0.5 * _build_a1(w1)).astype(bf16)
    b1r = jnp.broadcast_to((0.5 * b1).astype(f32).reshape(1, 1, 6, 1),
                           (4, 2, 6, 16)).reshape(768, 1)
    a2 = (0.25 * _build_a2(w2)).astype(bf16)
    b2v = 0.5 * (b2 + 0.5 * w2.sum(axis=(1, 2, 3)))
    b2r = jnp.broadcast_to(b2v.astype(f32).reshape(1, 1, 16, 1),
                           (2, 2, 16, 5)).reshape(320, 1)

    w3n = (0.0625 * jnp.transpose(w3.reshape(120, 16, 5, 5).astype(f32),
                                  (0, 2, 1, 3)).reshape(120, 400)).astype(bf16)
    b3m = (0.5 * b3 + 0.25 * w3.sum(axis=1)).reshape(120, 1).astype(f32)
    w4b = (0.25 * w4).astype(bf16)
    b4m = (0.5 * b4 + 0.25 * w4.sum(axis=1)).reshape(84, 1).astype(f32)
    w5b = (0.5 * w5).astype(bf16)
    b5m = (b5 + 0.5 * w5.sum(axis=1)).reshape(10, 1).astype(f32)

    def vmem_const(a):
        return pl.BlockSpec(a.shape, lambda g, nd=a.ndim: (0,) * nd)

    out = pl.pallas_call(
        _lenet_body,
        out_shape=jax.ShapeDtypeStruct((10, Bp), f32),
        grid=(G,),
        in_specs=[
            pl.BlockSpec((tb, 1024), lambda g: (g, 0)),
            vmem_const(a1), vmem_const(b1r),
            vmem_const(a2), vmem_const(b2r),
            vmem_const(w3n), vmem_const(b3m),
            vmem_const(w4b), vmem_const(b4m),
            vmem_const(w5b), vmem_const(b5m),
        ],
        out_specs=pl.BlockSpec((10, tb), lambda g: (0, g)),
        scratch_shapes=[
            pltpu.VMEM((1344, tb), bf16),
            pltpu.VMEM((400, tb), bf16),
        ],
        compiler_params=pltpu.CompilerParams(
            dimension_semantics=("parallel",),
            vmem_limit_bytes=48 * 1024 * 1024,
        ),
    )(x4, a1, b1r, a2, b2r, w3n, b3m, w4b, b4m, w5b, b5m)

    return out[:, :B].T


# speedup vs baseline: 1.0786x; 1.0028x over previous
import numpy as np

import jax
import jax.numpy as jnp
from jax.experimental import pallas as pl
from jax.experimental.pallas import tpu as pltpu

_TB = 4096


def _sig(v):
    return 0.5 * jnp.tanh(0.5 * v) + 0.5


def _conv1_maps():
    idx = np.zeros((768, 256), np.int32)
    msk = np.zeros((768, 256), bool)
    for dh in range(4):
        for p in range(2):
            for co in range(6):
                for wh in range(14):
                    row = dh * 192 + p * 96 + co * 16 + wh
                    for dy in range(5):
                        for dx in range(5):
                            col = (dh + dy) * 32 + (2 * wh + p + dx)
                            idx[row, col] = co * 25 + dy * 5 + dx
                            msk[row, col] = True
    co_of_row = np.zeros((768,), np.int32)
    for dh in range(4):
        for p in range(2):
            for co in range(6):
                for wh in range(16):
                    co_of_row[dh * 192 + p * 96 + co * 16 + wh] = co
    return idx, msk, co_of_row


def _conv2_maps():
    idx = np.zeros((320, 576), np.int32)
    msk = np.zeros((320, 576), bool)
    for dh in range(2):
        for p in range(2):
            for co in range(16):
                for wp in range(5):
                    row = dh * 160 + p * 80 + co * 5 + wp
                    for dy in range(5):
                        for ci in range(6):
                            for dx in range(5):
                                col = (dh + dy) * 96 + ci * 16 + (2 * wp + p + dx)
                                idx[row, col] = ((co * 6 + ci) * 5 + dy) * 5 + dx
                                msk[row, col] = True
    co_of_row = np.zeros((320,), np.int32)
    for dh in range(2):
        for p in range(2):
            for co in range(16):
                for wp in range(5):
                    co_of_row[dh * 160 + p * 80 + co * 5 + wp] = co
    return idx, msk, co_of_row


_A1_IDX, _A1_MSK, _B1_ROW = _conv1_maps()
_A2_IDX, _A2_MSK, _B2_ROW = _conv2_maps()


def _shift_rows(base, nrows, width):
    tiled = jnp.tile(base, (1,) * (base.ndim - 1) + (nrows,))
    return tiled[..., : nrows * width].reshape(*base.shape[:-1], nrows, width)


def _build_a1(w1):
    f32 = jnp.float32
    w = w1.reshape(6, 5, 5).astype(f32)
    parts = []
    for p in range(2):
        base = jnp.pad(w, ((0, 0), (0, 0), (p, 29 - p)))
        t = _shift_rows(base, 16, 32)
        parts.append(t)
    t1 = jnp.stack(parts, axis=0)
    whm = jnp.asarray((np.arange(16) < 14).astype(np.float32)).reshape(1, 1, 1, 16, 1)
    t1 = t1 * whm
    t1 = jnp.transpose(t1, (0, 1, 3, 4, 2))
    t1 = jnp.pad(t1, ((0, 0),) * 4 + ((0, 4),))
    t1 = _shift_rows(t1, 4, 8)
    t1 = jnp.transpose(t1, (4, 0, 1, 2, 5, 3))
    return t1.reshape(768, 256)


def _build_a2(w2):
    f32 = jnp.float32
    w = w2.astype(f32)
    parts = []
    for p in range(2):
        base = jnp.pad(w, ((0, 0),) * 3 + ((p, 13 - p),))
        t = _shift_rows(base, 5, 16)
        parts.append(t)
    t2 = jnp.stack(parts, axis=0)
    t2 = jnp.transpose(t2, (0, 1, 2, 4, 5, 3))
    t2 = jnp.pad(t2, ((0, 0),) * 5 + ((0, 2),))
    t2 = _shift_rows(t2, 2, 6)
    t2 = jnp.transpose(t2, (5, 0, 1, 3, 6, 2, 4))
    return 0.25 * t2.reshape(320, 576)


def _lenet_body(x_ref, a1_ref, b1_ref, a2_ref, b2_ref,
                w3_ref, b3_ref, w4_ref, b4_ref, w5_ref, b5_ref,
                out_ref, p1_ref, feat_ref):
    f32 = jnp.float32
    bf16 = jnp.bfloat16

    for c in range(7):
        xs = x_ref[:, 128 * c:128 * c + 256]
        acc = jax.lax.dot_general(
            a1_ref[...], xs, (((1,), (1,)), ((), ())),
            preferred_element_type=f32)
        s = jnp.tanh(acc + b1_ref[...])
        s4 = s.reshape(2, 2, 2, 96, _TB)
        pooled = s4[:, 0, 0] + s4[:, 0, 1] + s4[:, 1, 0] + s4[:, 1, 1]
        p1_ref[192 * c:192 * c + 192, :] = (
            pooled.reshape(192, _TB).astype(bf16))

    for c in range(5):
        ps = p1_ref[192 * c:192 * c + 576, :]
        acc = jnp.dot(a2_ref[...], ps, preferred_element_type=f32)
        s = jnp.tanh(acc + b2_ref[...])
        s4 = s.reshape(2, 2, 80, _TB)
        pooled = s4[0, 0] + s4[0, 1] + s4[1, 0] + s4[1, 1]
        feat_ref[80 * c:80 * c + 80, :] = pooled.astype(bf16)

    ft = feat_ref[...]
    h1 = jnp.tanh(jnp.dot(w3_ref[...], ft, preferred_element_type=f32) + b3_ref[...])
    h2 = jnp.tanh(jnp.dot(w4_ref[...], h1.astype(bf16),
                          preferred_element_type=f32) + b4_ref[...])
    out_ref[...] = jnp.dot(w5_ref[...], h2.astype(bf16),
                           preferred_element_type=f32) + b5_ref[...]


def kernel(x, w1, b1, w2, b2, w3, b3, w4, b4, w5, b5):
    f32, bf16 = jnp.float32, jnp.bfloat16
    tb = _TB
    B = int(np.prod(x.shape)) // 784
    G = pl.cdiv(B, tb)
    Bp = G * tb

    x2 = x.reshape(B, 28, 28).astype(bf16)
    x4 = jnp.pad(x2, ((0, Bp - B), (2, 2), (2, 2))).reshape(Bp, 1024)

    a1 = (0.5 * _build_a1(w1)).astype(bf16)
    b1r = jnp.broadcast_to((0.5 * b1).astype(f32).reshape(1, 1, 6, 1),
                           (4, 2, 6, 16)).reshape(768, 1)
    a2 = (0.25 * _build_a2(w2)).astype(bf16)
    b2v = 0.5 * (b2 + 0.5 * w2.sum(axis=(1, 2, 3)))
    b2r = jnp.broadcast_to(b2v.astype(f32).reshape(1, 1, 16, 1),
                           (2, 2, 16, 5)).reshape(320, 1)

    w3n = (0.0625 * jnp.transpose(w3.reshape(120, 16, 5, 5).astype(f32),
                                  (0, 2, 1, 3)).reshape(120, 400)).astype(bf16)
    b3m = (0.5 * b3 + 0.25 * w3.sum(axis=1)).reshape(120, 1).astype(f32)
    w4b = (0.25 * w4).astype(bf16)
    b4m = (0.5 * b4 + 0.25 * w4.sum(axis=1)).reshape(84, 1).astype(f32)
    w5b = (0.5 * w5).astype(bf16)
    b5m = (b5 + 0.5 * w5.sum(axis=1)).reshape(10, 1).astype(f32)

    def vmem_const(a):
        return pl.BlockSpec(a.shape, lambda g, nd=a.ndim: (0,) * nd)

    out = pl.pallas_call(
        _lenet_body,
        out_shape=jax.ShapeDtypeStruct((10, Bp), f32),
        grid=(G,),
        in_specs=[
            pl.BlockSpec((tb, 1024), lambda g: (g, 0)),
            vmem_const(a1), vmem_const(b1r),
            vmem_const(a2), vmem_const(b2r),
            vmem_const(w3n), vmem_const(b3m),
            vmem_const(w4b), vmem_const(b4m),
            vmem_const(w5b), vmem_const(b5m),
        ],
        out_specs=pl.BlockSpec((10, tb), lambda g: (0, g)),
        scratch_shapes=[
            pltpu.VMEM((1344, tb), bf16),
            pltpu.VMEM((400, tb), bf16),
        ],
        compiler_params=pltpu.CompilerParams(
            dimension_semantics=("parallel",),
            vmem_limit_bytes=48 * 1024 * 1024,
        ),
    )(x4, a1, b1r, a2, b2r, w3n, b3m, w4b, b4m, w5b, b5m)

    return out[:, :B].T


# speedup vs baseline: 1.0790x; 1.0004x over previous
import numpy as np

import jax
import jax.numpy as jnp
from jax.experimental import pallas as pl
from jax.experimental.pallas import tpu as pltpu

_TB = 2048


def _sig(v):
    return 0.5 * jnp.tanh(0.5 * v) + 0.5


def _conv1_maps():
    idx = np.zeros((768, 256), np.int32)
    msk = np.zeros((768, 256), bool)
    for dh in range(4):
        for p in range(2):
            for co in range(6):
                for wh in range(14):
                    row = dh * 192 + p * 96 + co * 16 + wh
                    for dy in range(5):
                        for dx in range(5):
                            col = (dh + dy) * 32 + (2 * wh + p + dx)
                            idx[row, col] = co * 25 + dy * 5 + dx
                            msk[row, col] = True
    co_of_row = np.zeros((768,), np.int32)
    for dh in range(4):
        for p in range(2):
            for co in range(6):
                for wh in range(16):
                    co_of_row[dh * 192 + p * 96 + co * 16 + wh] = co
    return idx, msk, co_of_row


def _conv2_maps():
    idx = np.zeros((320, 576), np.int32)
    msk = np.zeros((320, 576), bool)
    for dh in range(2):
        for p in range(2):
            for co in range(16):
                for wp in range(5):
                    row = dh * 160 + p * 80 + co * 5 + wp
                    for dy in range(5):
                        for ci in range(6):
                            for dx in range(5):
                                col = (dh + dy) * 96 + ci * 16 + (2 * wp + p + dx)
                                idx[row, col] = ((co * 6 + ci) * 5 + dy) * 5 + dx
                                msk[row, col] = True
    co_of_row = np.zeros((320,), np.int32)
    for dh in range(2):
        for p in range(2):
            for co in range(16):
                for wp in range(5):
                    co_of_row[dh * 160 + p * 80 + co * 5 + wp] = co
    return idx, msk, co_of_row


_A1_IDX, _A1_MSK, _B1_ROW = _conv1_maps()
_A2_IDX, _A2_MSK, _B2_ROW = _conv2_maps()


def _shift_rows(base, nrows, width):
    tiled = jnp.tile(base, (1,) * (base.ndim - 1) + (nrows,))
    return tiled[..., : nrows * width].reshape(*base.shape[:-1], nrows, width)


def _build_a1(w1):
    f32 = jnp.float32
    w = w1.reshape(6, 5, 5).astype(f32)
    parts = []
    for p in range(2):
        base = jnp.pad(w, ((0, 0), (0, 0), (p, 29 - p)))
        t = _shift_rows(base, 16, 32)
        parts.append(t)
    t1 = jnp.stack(parts, axis=0)
    whm = jnp.asarray((np.arange(16) < 14).astype(np.float32)).reshape(1, 1, 1, 16, 1)
    t1 = t1 * whm
    t1 = jnp.transpose(t1, (0, 1, 3, 4, 2))
    t1 = jnp.pad(t1, ((0, 0),) * 4 + ((0, 4),))
    t1 = _shift_rows(t1, 4, 8)
    t1 = jnp.transpose(t1, (4, 0, 1, 2, 5, 3))
    return t1.reshape(768, 256)


def _build_a2(w2):
    f32 = jnp.float32
    w = w2.astype(f32)
    parts = []
    for p in range(2):
        base = jnp.pad(w, ((0, 0),) * 3 + ((p, 13 - p),))
        t = _shift_rows(base, 5, 16)
        parts.append(t)
    t2 = jnp.stack(parts, axis=0)
    t2 = jnp.transpose(t2, (0, 1, 2, 4, 5, 3))
    t2 = jnp.pad(t2, ((0, 0),) * 5 + ((0, 2),))
    t2 = _shift_rows(t2, 2, 6)
    t2 = jnp.transpose(t2, (5, 0, 1, 3, 6, 2, 4))
    return 0.25 * t2.reshape(320, 576)


def _lenet_body(x_ref, a1_ref, b1_ref, a2_ref, b2_ref,
                w3_ref, b3_ref, w4_ref, b4_ref, w5_ref, b5_ref,
                out_ref, p1_ref, feat_ref):
    f32 = jnp.float32
    bf16 = jnp.bfloat16

    def conv1_chunk(c):
        xs = x_ref[:, 128 * c:128 * c + 256]
        acc = jax.lax.dot_general(
            a1_ref[...], xs, (((1,), (1,)), ((), ())),
            preferred_element_type=f32)
        s = jnp.tanh(acc + b1_ref[...])
        s4 = s.reshape(2, 2, 2, 96, _TB)
        pooled = s4[:, 0, 0] + s4[:, 0, 1] + s4[:, 1, 0] + s4[:, 1, 1]
        p1_ref[192 * c:192 * c + 192, :] = (
            pooled.reshape(192, _TB).astype(bf16))

    def conv2_chunk(c):
        ps = p1_ref[192 * c:192 * c + 576, :]
        acc = jnp.dot(a2_ref[...], ps, preferred_element_type=f32)
        s = jnp.tanh(acc + b2_ref[...])
        s4 = s.reshape(2, 2, 80, _TB)
        pooled = s4[0, 0] + s4[0, 1] + s4[1, 0] + s4[1, 1]
        feat_ref[80 * c:80 * c + 80, :] = pooled.astype(bf16)

    for c in range(3):
        conv1_chunk(c)
    for c in range(5):
        if c + 3 < 7:
            conv1_chunk(c + 3)
        conv2_chunk(c)

    ft = feat_ref[...]
    h1 = jnp.tanh(jnp.dot(w3_ref[...], ft, preferred_element_type=f32) + b3_ref[...])
    h2 = jnp.tanh(jnp.dot(w4_ref[...], h1.astype(bf16),
                          preferred_element_type=f32) + b4_ref[...])
    out_ref[...] = jnp.dot(w5_ref[...], h2.astype(bf16),
                           preferred_element_type=f32) + b5_ref[...]


def kernel(x, w1, b1, w2, b2, w3, b3, w4, b4, w5, b5):
    f32, bf16 = jnp.float32, jnp.bfloat16
    tb = _TB
    B = int(np.prod(x.shape)) // 784
    G = pl.cdiv(B, tb)
    Bp = G * tb

    x2 = x.reshape(B, 28, 28).astype(bf16)
    x4 = jnp.pad(x2, ((0, Bp - B), (2, 2), (2, 2))).reshape(Bp, 1024)

    a1 = (0.5 * _build_a1(w1)).astype(bf16)
    b1r = jnp.broadcast_to((0.5 * b1).astype(f32).reshape(1, 1, 6, 1),
                           (4, 2, 6, 16)).reshape(768, 1)
    a2 = (0.25 * _build_a2(w2)).astype(bf16)
    b2v = 0.5 * (b2 + 0.5 * w2.sum(axis=(1, 2, 3)))
    b2r = jnp.broadcast_to(b2v.astype(f32).reshape(1, 1, 16, 1),
                           (2, 2, 16, 5)).reshape(320, 1)

    w3n = (0.0625 * jnp.transpose(w3.reshape(120, 16, 5, 5).astype(f32),
                                  (0, 2, 1, 3)).reshape(120, 400)).astype(bf16)
    b3m = (0.5 * b3 + 0.25 * w3.sum(axis=1)).reshape(120, 1).astype(f32)
    w4b = (0.25 * w4).astype(bf16)
    b4m = (0.5 * b4 + 0.25 * w4.sum(axis=1)).reshape(84, 1).astype(f32)
    w5b = (0.5 * w5).astype(bf16)
    b5m = (b5 + 0.5 * w5.sum(axis=1)).reshape(10, 1).astype(f32)

    def vmem_const(a):
        return pl.BlockSpec(a.shape, lambda g, nd=a.ndim: (0,) * nd)

    out = pl.pallas_call(
        _lenet_body,
        out_shape=jax.ShapeDtypeStruct((10, Bp), f32),
        grid=(G,),
        in_specs=[
            pl.BlockSpec((tb, 1024), lambda g: (g, 0)),
            vmem_const(a1), vmem_const(b1r),
            vmem_const(a2), vmem_const(b2r),
            vmem_const(w3n), vmem_const(b3m),
            vmem_const(w4b), vmem_const(b4m),
            vmem_const(w5b), vmem_const(b5m),
        ],
        out_specs=pl.BlockSpec((10, tb), lambda g: (0, g)),
        scratch_shapes=[
            pltpu.VMEM((1344, tb), bf16),
            pltpu.VMEM((400, tb), bf16),
        ],
        compiler_params=pltpu.CompilerParams(
            dimension_semantics=("parallel",),
            vmem_limit_bytes=48 * 1024 * 1024,
        ),
    )(x4, a1, b1r, a2, b2r, w3n, b3m, w4b, b4m, w5b, b5m)

    return out[:, :B].T


# speedup vs baseline: 1.0802x; 1.0011x over previous
import numpy as np

import jax
import jax.numpy as jnp
from jax.experimental import pallas as pl
from jax.experimental.pallas import tpu as pltpu

_TB = 2048


def _shift_rows(base, nrows, width):
    tiled = jnp.tile(base, (1,) * (base.ndim - 1) + (nrows,))
    return tiled[..., : nrows * width].reshape(*base.shape[:-1], nrows, width)


def _build_a1(w1):
    f32 = jnp.float32
    w = w1.reshape(6, 5, 5).astype(f32)
    parts = []
    for p in range(2):
        base = jnp.pad(w, ((0, 0), (0, 0), (p, 29 - p)))
        t = _shift_rows(base, 16, 32)
        parts.append(t)
    t1 = jnp.stack(parts, axis=0)
    whm = jnp.asarray((np.arange(16) < 14).astype(np.float32)).reshape(1, 1, 1, 16, 1)
    t1 = t1 * whm
    t1 = jnp.transpose(t1, (0, 1, 3, 4, 2))
    t1 = jnp.pad(t1, ((0, 0),) * 4 + ((0, 4),))
    t1 = _shift_rows(t1, 4, 8)
    t1 = jnp.transpose(t1, (4, 0, 1, 2, 5, 3))
    return t1.reshape(768, 256)


def _build_a2(w2):
    f32 = jnp.float32
    w = w2.astype(f32)
    parts = []
    for p in range(2):
        base = jnp.pad(w, ((0, 0),) * 3 + ((p, 13 - p),))
        t = _shift_rows(base, 5, 16)
        parts.append(t)
    t2 = jnp.stack(parts, axis=0)
    t2 = jnp.transpose(t2, (0, 1, 2, 4, 5, 3))
    t2 = jnp.pad(t2, ((0, 0),) * 5 + ((0, 2),))
    t2 = _shift_rows(t2, 2, 6)
    t2 = jnp.transpose(t2, (5, 0, 1, 3, 6, 2, 4))
    return 0.25 * t2.reshape(320, 576)


def _lenet_body(x_ref, a1_ref, b1_ref, a2_ref, b2_ref,
                w3_ref, b3_ref, w4_ref, b4_ref, w5_ref, b5_ref,
                out_ref, p1_ref, feat_ref):
    f32 = jnp.float32
    bf16 = jnp.bfloat16

    def conv1_chunk(c):
        xs = x_ref[:, 128 * c:128 * c + 256]
        acc = jax.lax.dot_general(
            a1_ref[...], xs, (((1,), (1,)), ((), ())),
            preferred_element_type=f32)
        s = jnp.tanh(acc + b1_ref[...])
        s4 = s.reshape(2, 2, 2, 96, _TB)
        pooled = s4[:, 0, 0] + s4[:, 0, 1] + s4[:, 1, 0] + s4[:, 1, 1]
        p1_ref[192 * c:192 * c + 192, :] = (
            pooled.reshape(192, _TB).astype(bf16))

    def conv2_chunk(c):
        ps = p1_ref[192 * c:192 * c + 576, :]
        acc = jnp.dot(a2_ref[...], ps, preferred_element_type=f32)
        s = jnp.tanh(acc + b2_ref[...])
        s4 = s.reshape(2, 2, 80, _TB)
        pooled = s4[0, 0] + s4[0, 1] + s4[1, 0] + s4[1, 1]
        feat_ref[80 * c:80 * c + 80, :] = pooled.astype(bf16)

    for c in range(3):
        conv1_chunk(c)
    for c in range(5):
        if c + 3 < 7:
            conv1_chunk(c + 3)
        conv2_chunk(c)

    ft = feat_ref[...]
    h1 = jnp.tanh(jnp.dot(w3_ref[...], ft, preferred_element_type=f32) + b3_ref[...])
    h2 = jnp.tanh(jnp.dot(w4_ref[...], h1.astype(bf16),
                          preferred_element_type=f32) + b4_ref[...])
    out_ref[...] = jnp.dot(w5_ref[...], h2.astype(bf16),
                           preferred_element_type=f32) + b5_ref[...]


def kernel(x, w1, b1, w2, b2, w3, b3, w4, b4, w5, b5):
    f32, bf16 = jnp.float32, jnp.bfloat16
    tb = _TB
    B = int(np.prod(x.shape)) // 784
    G = pl.cdiv(B, tb)
    Bp = G * tb

    x2 = x.reshape(B, 28, 28).astype(bf16)
    x4 = jnp.pad(x2, ((0, Bp - B), (2, 2), (2, 2))).reshape(Bp, 1024)

    a1 = (0.5 * _build_a1(w1)).astype(bf16)
    b1r = jnp.broadcast_to((0.5 * b1).astype(f32).reshape(1, 1, 6, 1),
                           (4, 2, 6, 16)).reshape(768, 1)
    a2 = (0.25 * _build_a2(w2)).astype(bf16)
    b2v = 0.5 * (b2 + 0.5 * w2.sum(axis=(1, 2, 3)))
    b2r = jnp.broadcast_to(b2v.astype(f32).reshape(1, 1, 16, 1),
                           (2, 2, 16, 5)).reshape(320, 1)

    w3n = (0.0625 * jnp.transpose(w3.reshape(120, 16, 5, 5).astype(f32),
                                  (0, 2, 1, 3)).reshape(120, 400)).astype(bf16)
    b3m = (0.5 * b3 + 0.25 * w3.sum(axis=1)).reshape(120, 1).astype(f32)
    w4b = (0.25 * w4).astype(bf16)
    b4m = (0.5 * b4 + 0.25 * w4.sum(axis=1)).reshape(84, 1).astype(f32)
    w5b = (0.5 * w5).astype(bf16)
    b5m = (b5 + 0.5 * w5.sum(axis=1)).reshape(10, 1).astype(f32)

    def vmem_const(a):
        return pl.BlockSpec(a.shape, lambda g, nd=a.ndim: (0,) * nd)

    out = pl.pallas_call(
        _lenet_body,
        out_shape=jax.ShapeDtypeStruct((10, Bp), f32),
        grid=(G,),
        in_specs=[
            pl.BlockSpec((tb, 1024), lambda g: (g, 0)),
            vmem_const(a1), vmem_const(b1r),
            vmem_const(a2), vmem_const(b2r),
            vmem_const(w3n), vmem_const(b3m),
            vmem_const(w4b), vmem_const(b4m),
            vmem_const(w5b), vmem_const(b5m),
        ],
        out_specs=pl.BlockSpec((10, tb), lambda g: (0, g)),
        scratch_shapes=[
            pltpu.VMEM((1344, tb), bf16),
            pltpu.VMEM((400, tb), bf16),
        ],
        compiler_params=pltpu.CompilerParams(
            dimension_semantics=("parallel",),
            vmem_limit_bytes=48 * 1024 * 1024,
        ),
    )(x4, a1, b1r, a2, b2r, w3n, b3m, w4b, b4m, w5b, b5m)

    return out[:, :B].T
